# baseline (speedup 1.0000x reference)
import functools

import jax
import jax.numpy as jnp
from jax import lax
from jax.experimental import pallas as pl
from jax.experimental.pallas import tpu as pltpu
from jax.experimental.pallas import tpu_sc as plsc

CHUNK = 128


def _gather_rows(table, idx3, n_workers, n_chunks, d):
  npad = n_workers * n_chunks * CHUNK
  mesh = plsc.VectorSubcoreMesh(core_axis_name="c", subcore_axis_name="s")
  info = plsc.get_sparse_core_info()

  @functools.partial(
      pl.kernel,
      mesh=mesh,
      out_type=jax.ShapeDtypeStruct((npad, d), jnp.float32),
      compiler_params=pltpu.CompilerParams(use_tc_tiling_on_sc=False),
      scratch_types=[
          pltpu.VMEM((n_chunks, CHUNK), jnp.int32),
          pltpu.VMEM((CHUNK, d), jnp.float32),
          pltpu.SemaphoreType.DMA,
      ],
  )
  def k(table_hbm, idx_hbm, out_hbm, idx_v, rows_v, sem):
    wid = lax.axis_index("s") * info.num_cores + lax.axis_index("c")
    pltpu.sync_copy(idx_hbm.at[wid], idx_v)
    base = wid * (n_chunks * CHUNK)

    def body(j, carry):
      pltpu.async_copy(table_hbm.at[idx_v.at[j]], rows_v, sem).wait()
      pltpu.sync_copy(rows_v, out_hbm.at[pl.ds(base + j * CHUNK, CHUNK)])
      return carry

    lax.fori_loop(0, n_chunks, body, 0)

  return k(table, idx3)


def kernel(table, indices):
  b, s = indices.shape
  v, d = table.shape
  n = b * s
  idx = indices.reshape(n).astype(jnp.int32)

  info = plsc.get_sparse_core_info()
  n_workers = info.num_cores * info.num_subcores
  per = n_workers * CHUNK
  npad = ((n + per - 1) // per) * per
  if npad != n:
    idx = jnp.concatenate([idx, jnp.zeros((npad - n,), jnp.int32)])
  n_chunks = npad // per
  idx3 = idx.reshape(n_workers, n_chunks, CHUNK)

  out = _gather_rows(table, idx3, n_workers, n_chunks, d)
  return out[:n].reshape(b, s, d)


# speedup vs baseline: 1.1139x; 1.1139x over previous
import functools

import jax
import jax.numpy as jnp
from jax import lax
from jax.experimental import pallas as pl
from jax.experimental.pallas import tpu as pltpu
from jax.experimental.pallas import tpu_sc as plsc

CHUNK = 256
NBUF = 4


def _gather_rows(table, idx2, n_workers, n_chunks, d):
  npad = n_workers * n_chunks * CHUNK
  per_w = n_chunks * CHUNK
  mesh = plsc.VectorSubcoreMesh(core_axis_name="c", subcore_axis_name="s")
  info = plsc.get_sparse_core_info()

  @functools.partial(
      pl.kernel,
      mesh=mesh,
      out_type=jax.ShapeDtypeStruct((npad, d), jnp.float32),
      compiler_params=pltpu.CompilerParams(use_tc_tiling_on_sc=False),
      scratch_types=[
          pltpu.VMEM((per_w,), jnp.int32),
          pltpu.VMEM((NBUF, CHUNK, d), jnp.float32),
          pltpu.SemaphoreType.DMA((NBUF,)),
          pltpu.SemaphoreType.DMA((NBUF,)),
      ],
  )
  def k(table_hbm, idx_hbm, out_hbm, idx_v, rows_v, gsem, wsem):
    wid = lax.axis_index("s") * info.num_cores + lax.axis_index("c")
    pltpu.sync_copy(idx_hbm.at[wid], idx_v)
    base = wid * per_w

    def gfire(j, b):
      pltpu.async_copy(
          table_hbm.at[idx_v.at[pl.ds(j * CHUNK, CHUNK)]],
          rows_v.at[b], gsem.at[b])

    def gwait(j, b):
      pltpu.make_async_copy(
          table_hbm.at[idx_v.at[pl.ds(j * CHUNK, CHUNK)]],
          rows_v.at[b], gsem.at[b]).wait()

    def wfire(j, b):
      pltpu.async_copy(
          rows_v.at[b], out_hbm.at[pl.ds(base + j * CHUNK, CHUNK)], wsem.at[b])

    def wwait(j, b):
      pltpu.make_async_copy(
          rows_v.at[b], out_hbm.at[pl.ds(base + j * CHUNK, CHUNK)],
          wsem.at[b]).wait()

    for b in range(NBUF):
      gfire(b, b)

    def body(i, carry):
      for b in range(NBUF):
        j = i * NBUF + b
        gwait(j, b)
        wfire(j, b)
        wwait(j, b)
        gfire(j + NBUF, b)
      return carry

    lax.fori_loop(0, n_chunks // NBUF - 1, body, 0)

    for b in range(NBUF):
      j = n_chunks - NBUF + b
      gwait(j, b)
      wfire(j, b)
      wwait(j, b)

  return k(table, idx2)


def kernel(table, indices):
  b, s = indices.shape
  v, d = table.shape
  n = b * s
  idx = indices.reshape(n).astype(jnp.int32)

  info = plsc.get_sparse_core_info()
  n_workers = info.num_cores * info.num_subcores
  per = n_workers * CHUNK * NBUF
  npad = ((n + per - 1) // per) * per
  if npad != n:
    idx = jnp.concatenate([idx, jnp.zeros((npad - n,), jnp.int32)])
  n_chunks = npad // (n_workers * CHUNK)
  idx2 = idx.reshape(n_workers, n_chunks * CHUNK)

  out = _gather_rows(table, idx2, n_workers, n_chunks, d)
  return out[:n].reshape(b, s, d)


# speedup vs baseline: 1.1444x; 1.0274x over previous
import functools

import jax
import jax.numpy as jnp
from jax import lax
from jax.experimental import pallas as pl
from jax.experimental.pallas import tpu as pltpu
from jax.experimental.pallas import tpu_sc as plsc

CHUNK = 256
NBUF = 4


def _gather_rows(table, idx2, n_workers, n_chunks, d):
  npad = n_workers * n_chunks * CHUNK
  per_w = n_chunks * CHUNK
  mesh = plsc.VectorSubcoreMesh(core_axis_name="c", subcore_axis_name="s")
  info = plsc.get_sparse_core_info()

  @functools.partial(
      pl.kernel,
      mesh=mesh,
      out_type=jax.ShapeDtypeStruct((npad, d), jnp.float32),
      compiler_params=pltpu.CompilerParams(use_tc_tiling_on_sc=False),
      scratch_types=[
          pltpu.VMEM((per_w,), jnp.int32),
          pltpu.VMEM((NBUF, CHUNK, d), jnp.float32),
          pltpu.SemaphoreType.DMA((NBUF,)),
          pltpu.SemaphoreType.DMA((NBUF,)),
      ],
  )
  def k(table_hbm, idx_hbm, out_hbm, idx_v, rows_v, gsem, wsem):
    wid = lax.axis_index("s") * info.num_cores + lax.axis_index("c")
    pltpu.sync_copy(idx_hbm.at[wid], idx_v)
    base = wid * per_w

    def gfire(j, b):
      pltpu.async_copy(
          table_hbm.at[idx_v.at[pl.ds(j * CHUNK, CHUNK)]],
          rows_v.at[b], gsem.at[b])

    def gwait(j, b):
      pltpu.make_async_copy(
          table_hbm.at[idx_v.at[pl.ds(j * CHUNK, CHUNK)]],
          rows_v.at[b], gsem.at[b]).wait()

    def wfire(j, b):
      pltpu.async_copy(
          rows_v.at[b], out_hbm.at[pl.ds(base + j * CHUNK, CHUNK)], wsem.at[b])

    def wwait(j, b):
      pltpu.make_async_copy(
          rows_v.at[b], out_hbm.at[pl.ds(base + j * CHUNK, CHUNK)],
          wsem.at[b]).wait()

    for b in range(NBUF):
      gfire(b, b)

    def body(i, carry):
      for b in range(NBUF):
        j = i * NBUF + b
        gwait(j, b)
        wfire(j, b)
        wwait(j, b)
        gfire(j + NBUF, b)
      return carry

    lax.fori_loop(0, n_chunks // NBUF - 1, body, 0)

    for b in range(NBUF):
      j = n_chunks - NBUF + b
      gwait(j, b)
      wfire(j, b)
      wwait(j, b)

  return k(table, idx2)


def kernel(table, indices):
  b, s = indices.shape
  v, d = table.shape
  n = b * s
  idx = indices.T.reshape(n).astype(jnp.int32)

  info = plsc.get_sparse_core_info()
  n_workers = info.num_cores * info.num_subcores
  per = n_workers * CHUNK * NBUF
  npad = ((n + per - 1) // per) * per
  if npad != n:
    idx = jnp.concatenate([idx, jnp.zeros((npad - n,), jnp.int32)])
  n_chunks = npad // (n_workers * CHUNK)
  idx2 = idx.reshape(n_workers, n_chunks * CHUNK)

  out = _gather_rows(table, idx2, n_workers, n_chunks, d)
  return out[:n].reshape(s, b, d).transpose(1, 0, 2)
